# baseline (speedup 1.0000x reference)
import functools

import jax
import jax.numpy as jnp
from jax import lax
from jax.experimental import pallas as pl
from jax.experimental.pallas import tpu as pltpu
from jax.experimental.pallas import tpu_sc as plsc

_M = 100000
_D = 64
_C = 4096
_B = 1024
_TEMP = 0.05

_NW = 32
_CHUNK = 128
_NFULL = _M // _CHUNK
_TAIL = _M - _NFULL * _CHUNK
_JMAX = (_NFULL + _NW - 1) // _NW
_NB = 2
_ZROWS = _C // 16
_CW = 16


def _sc_segment_sum(features, labels, lbl2d, indexes, zeros_d, zeros_c,
                    ones_c):
  mesh = plsc.VectorSubcoreMesh(core_axis_name="c", subcore_axis_name="s")

  @functools.partial(
      pl.kernel,
      out_type=[
          jax.ShapeDtypeStruct((2, _C, _D), jnp.float32),
          jax.ShapeDtypeStruct((2, _C, _CW), jnp.float32),
          jax.ShapeDtypeStruct((_B,), jnp.int32),
      ],
      mesh=mesh,
      scratch_types=[
          pltpu.VMEM((_JMAX, _CHUNK), jnp.int32),
          pltpu.VMEM((_NB, _CHUNK, _D), jnp.float32),
          pltpu.VMEM((_CHUNK, _CW), jnp.float32),
          pltpu.VMEM((1, _TAIL), jnp.int32),
          pltpu.VMEM((_TAIL, _D), jnp.float32),
          pltpu.VMEM((_CHUNK,), jnp.int32),
          pltpu.VMEM((_CHUNK,), jnp.int32),
          pltpu.SemaphoreType.DMA((_NB,)),
          pltpu.SemaphoreType.DMA,
          pltpu.VMEM_SHARED((_C, _D), jnp.float32),
          pltpu.VMEM_SHARED((_C, _CW), jnp.float32),
      ],
  )
  def k(feat_hbm, lbl_hbm, lbl2d_hbm, idx_hbm, zd_hbm, zc_hbm, ones_hbm,
        sums_out, cnts_out, tgt_out,
        lbl_a, feat_v, ones_v, tl_v, tf_v, idx_v, tgt_v,
        fsem, csem, acc_s, cnt_s):
    cid = lax.axis_index("c")
    sid = lax.axis_index("s")
    wid = sid * 2 + cid

    base = 24 * wid
    nw = 24 + (wid < 13).astype(jnp.int32)

    pltpu.make_async_copy(
        lbl2d_hbm.at[pl.ds(base, 24)], lbl_a.at[pl.ds(0, 24)],
        fsem.at[0]).start()
    pltpu.sync_copy(zd_hbm, acc_s.at[pl.ds(sid * _ZROWS, _ZROWS)])
    pltpu.sync_copy(zc_hbm, cnt_s.at[pl.ds(sid * _ZROWS, _ZROWS)])
    pltpu.sync_copy(ones_hbm, ones_v)
    pltpu.make_async_copy(
        lbl2d_hbm.at[pl.ds(base, 24)], lbl_a.at[pl.ds(0, 24)],
        fsem.at[0]).wait()

    @pl.when(wid < 13)
    def _():
      pltpu.sync_copy(lbl_hbm.at[pl.ds((768 + wid) * _CHUNK, _CHUNK)],
                      lbl_a.at[24])

    plsc.subcore_barrier()

    def valid(j):
      return (j >= 0) & (j < nw)

    def chunk_off(j):
      return jnp.where(j < 24, base + j, 768 + wid) * _CHUNK

    def start_load(j, b):
      pltpu.make_async_copy(
          feat_hbm.at[pl.ds(chunk_off(j), _CHUNK)], feat_v.at[b],
          fsem.at[b]).start()

    def wait_load(j, b):
      pltpu.make_async_copy(
          feat_hbm.at[pl.ds(chunk_off(j), _CHUNK)], feat_v.at[b],
          fsem.at[b]).wait()

    def do_scatter(j, b):
      pltpu.sync_copy(feat_v.at[b], acc_s.at[lbl_a.at[j]], add=True)
      pltpu.async_copy(ones_v, cnt_s.at[lbl_a.at[j]], csem, add=True)

    @pl.when(valid(0))
    def _():
      start_load(0, 0)

    def step(j, b):
      @pl.when(valid(j + 1))
      def _():
        start_load(j + 1, (b + 1) % _NB)

      @pl.when(valid(j))
      def _():
        wait_load(j, b)
        do_scatter(j, b)

    nsteps = _JMAX + _NB - 1
    ngroups = (nsteps + _NB - 1) // _NB

    def body(g, carry):
      for u in range(_NB):
        step(_NB * g + u, u)
      return carry

    lax.fori_loop(0, ngroups, body, 0)

    def drain(i, carry):
      pltpu.make_async_copy(ones_v, cnt_s.at[lbl_a.at[0]], csem).wait()
      return carry

    lax.fori_loop(0, nw, drain, 0)

    @pl.when(wid == _NW - 1)
    def _():
      off = _NFULL * _CHUNK
      pltpu.sync_copy(lbl_hbm.at[pl.ds(off, _TAIL)], tl_v.at[0])
      pltpu.sync_copy(feat_hbm.at[pl.ds(off, _TAIL)], tf_v)
      pltpu.sync_copy(tf_v, acc_s.at[tl_v.at[0]], add=True)
      pltpu.sync_copy(ones_v.at[pl.ds(0, _TAIL)], cnt_s.at[tl_v.at[0]],
                      add=True)

    plsc.subcore_barrier()

    row = pl.ds(sid * _ZROWS, _ZROWS)
    pltpu.sync_copy(acc_s.at[row], sums_out.at[cid].at[row])
    pltpu.sync_copy(cnt_s.at[row], cnts_out.at[cid].at[row])

    @pl.when(wid < _B // _CHUNK)
    def _():
      boff = wid * _CHUNK
      pltpu.sync_copy(idx_hbm.at[pl.ds(boff, _CHUNK)], idx_v)
      pltpu.sync_copy(lbl_hbm.at[idx_v], tgt_v)
      pltpu.sync_copy(tgt_v, tgt_out.at[pl.ds(boff, _CHUNK)])

  return k(features, labels, lbl2d, indexes, zeros_d, zeros_c, ones_c)


_CBLK = 512


def _tc_body(x_ref, s_ref, c_ref, t_ref, o_ref, rs_acc, tv_acc):
  i = pl.program_id(0)
  x = x_ref[...]
  nrm = jnp.sqrt(jnp.sum(x * x, axis=1, keepdims=True))
  xn = x / jnp.maximum(nrm, 1e-12)
  s = s_ref[...]
  f = s[0] + s[1]
  c = c_ref[...]
  cnt = c[0, :, 0] + c[1, :, 0]
  inv = 1.0 / (_TEMP * jnp.where(cnt > 0, cnt, 1.0))
  vec = lax.dot_general(xn, f * inv[:, None], (((1,), (1,)), ((), ())),
                        preferred_element_type=jnp.float32)
  e = jnp.exp(vec) * (cnt > 0).astype(jnp.float32)[None, :]
  colid = i * _CBLK + lax.broadcasted_iota(jnp.int32, (_B, _CBLK), 1)
  tmask = (colid == t_ref[...]).astype(jnp.float32)
  ps = jnp.sum(e, axis=1, keepdims=True)
  pt = jnp.sum(e * tmask, axis=1, keepdims=True)

  @pl.when(i == 0)
  def _():
    rs_acc[...] = ps
    tv_acc[...] = pt

  @pl.when(i > 0)
  def _():
    rs_acc[...] += ps
    tv_acc[...] += pt

  @pl.when(i == pl.num_programs(0) - 1)
  def _():
    tot = rs_acc[...] + 1e-6
    logp = jnp.log(tv_acc[...] / tot + 1e-6)
    o_ref[...] = jnp.mean(-logp).reshape(1, 1)


def _tc_loss(results, sums, cnts, targets):
  return pl.pallas_call(
      _tc_body,
      grid=(_C // _CBLK,),
      in_specs=[
          pl.BlockSpec((_B, _D), lambda i: (0, 0)),
          pl.BlockSpec((2, _CBLK, _D), lambda i: (0, i, 0)),
          pl.BlockSpec((2, _CBLK, _CW), lambda i: (0, i, 0)),
          pl.BlockSpec((_B, 1), lambda i: (0, 0)),
      ],
      out_specs=pl.BlockSpec((1, 1), lambda i: (0, 0)),
      out_shape=jax.ShapeDtypeStruct((1, 1), jnp.float32),
      scratch_shapes=[
          pltpu.VMEM((_B, 1), jnp.float32),
          pltpu.VMEM((_B, 1), jnp.float32),
      ],
  )(results, sums, cnts, targets.reshape(_B, 1))


def kernel(results, indexes, features, labels):
  zeros_d = jnp.zeros((_ZROWS, _D), jnp.float32)
  zeros_c = jnp.zeros((_ZROWS, _CW), jnp.float32)
  ones_c = jnp.ones((_CHUNK, _CW), jnp.float32)
  labels = labels.astype(jnp.int32)
  lbl2d = lax.slice(labels, (0,), (_NFULL * _CHUNK,)).reshape(_NFULL, _CHUNK)
  sums, cnts, targets = _sc_segment_sum(
      features, labels, lbl2d, indexes.astype(jnp.int32),
      zeros_d, zeros_c, ones_c)
  out = _tc_loss(results, sums, cnts, targets)
  return out[0, 0]


# speedup vs baseline: 1.0081x; 1.0081x over previous
import functools

import jax
import jax.numpy as jnp
from jax import lax
from jax.experimental import pallas as pl
from jax.experimental.pallas import tpu as pltpu
from jax.experimental.pallas import tpu_sc as plsc

_M = 100000
_D = 64
_C = 4096
_B = 1024
_TEMP = 0.05

_NW = 32
_CHUNK = 128
_NFULL = _M // _CHUNK
_TAIL = _M - _NFULL * _CHUNK
_JMAX = (_NFULL + _NW - 1) // _NW
_NB = 2
_ZROWS = _C // 16
_CW = 16


def _sc_segment_sum(features, labels, lbl2d, indexes, zeros_d, zeros_c,
                    ones_c):
  mesh = plsc.VectorSubcoreMesh(core_axis_name="c", subcore_axis_name="s")

  @functools.partial(
      pl.kernel,
      out_type=[
          jax.ShapeDtypeStruct((2, _C, _D), jnp.float32),
          jax.ShapeDtypeStruct((2, _C, _CW), jnp.float32),
          jax.ShapeDtypeStruct((_B,), jnp.int32),
      ],
      mesh=mesh,
      scratch_types=[
          pltpu.VMEM((_JMAX, _CHUNK), jnp.int32),
          pltpu.VMEM((_NB, _CHUNK, _D), jnp.float32),
          pltpu.VMEM((_CHUNK, _CW), jnp.float32),
          pltpu.VMEM((1, _TAIL), jnp.int32),
          pltpu.VMEM((_TAIL, _D), jnp.float32),
          pltpu.VMEM((_CHUNK,), jnp.int32),
          pltpu.VMEM((_CHUNK,), jnp.int32),
          pltpu.SemaphoreType.DMA((_NB,)),
          pltpu.VMEM_SHARED((_C, _D), jnp.float32),
          pltpu.VMEM_SHARED((_C, _CW), jnp.float32),
      ],
  )
  def k(feat_hbm, lbl_hbm, lbl2d_hbm, idx_hbm, zd_hbm, zc_hbm, ones_hbm,
        sums_out, cnts_out, tgt_out,
        lbl_a, feat_v, ones_v, tl_v, tf_v, idx_v, tgt_v,
        fsem, acc_s, cnt_s):
    cid = lax.axis_index("c")
    sid = lax.axis_index("s")
    wid = sid * 2 + cid

    base = 24 * wid
    nw = 24 + (wid < 13).astype(jnp.int32)

    pltpu.make_async_copy(
        lbl2d_hbm.at[pl.ds(base, 24)], lbl_a.at[pl.ds(0, 24)],
        fsem.at[0]).start()
    pltpu.sync_copy(zd_hbm, acc_s.at[pl.ds(sid * _ZROWS, _ZROWS)])
    pltpu.sync_copy(zc_hbm, cnt_s.at[pl.ds(sid * _ZROWS, _ZROWS)])
    pltpu.sync_copy(ones_hbm, ones_v)
    pltpu.make_async_copy(
        lbl2d_hbm.at[pl.ds(base, 24)], lbl_a.at[pl.ds(0, 24)],
        fsem.at[0]).wait()

    @pl.when(wid < 13)
    def _():
      pltpu.sync_copy(lbl_hbm.at[pl.ds((768 + wid) * _CHUNK, _CHUNK)],
                      lbl_a.at[24])

    plsc.subcore_barrier()

    def valid(j):
      return (j >= 0) & (j < nw)

    def chunk_off(j):
      return jnp.where(j < 24, base + j, 768 + wid) * _CHUNK

    def start_load(j, b):
      pltpu.make_async_copy(
          feat_hbm.at[pl.ds(chunk_off(j), _CHUNK)], feat_v.at[b],
          fsem.at[b]).start()

    def wait_load(j, b):
      pltpu.make_async_copy(
          feat_hbm.at[pl.ds(chunk_off(j), _CHUNK)], feat_v.at[b],
          fsem.at[b]).wait()

    def do_scatter(j, b):
      pltpu.sync_copy(feat_v.at[b], acc_s.at[lbl_a.at[j]], add=True)
      pltpu.sync_copy(ones_v, cnt_s.at[lbl_a.at[j]], add=True)

    @pl.when(valid(0))
    def _():
      start_load(0, 0)

    def step(j, b):
      @pl.when(valid(j + 1))
      def _():
        start_load(j + 1, (b + 1) % _NB)

      @pl.when(valid(j))
      def _():
        wait_load(j, b)
        do_scatter(j, b)

    nsteps = _JMAX + _NB - 1
    ngroups = (nsteps + _NB - 1) // _NB

    def body(g, carry):
      for u in range(_NB):
        step(_NB * g + u, u)
      return carry

    lax.fori_loop(0, ngroups, body, 0)

    @pl.when(wid == _NW - 1)
    def _():
      off = _NFULL * _CHUNK
      pltpu.sync_copy(lbl_hbm.at[pl.ds(off, _TAIL)], tl_v.at[0])
      pltpu.sync_copy(feat_hbm.at[pl.ds(off, _TAIL)], tf_v)
      pltpu.sync_copy(tf_v, acc_s.at[tl_v.at[0]], add=True)
      pltpu.sync_copy(ones_v.at[pl.ds(0, _TAIL)], cnt_s.at[tl_v.at[0]],
                      add=True)

    plsc.subcore_barrier()

    row = pl.ds(sid * _ZROWS, _ZROWS)
    pltpu.sync_copy(acc_s.at[row], sums_out.at[cid].at[row])
    pltpu.sync_copy(cnt_s.at[row], cnts_out.at[cid].at[row])

    @pl.when(wid < _B // _CHUNK)
    def _():
      boff = wid * _CHUNK
      pltpu.sync_copy(idx_hbm.at[pl.ds(boff, _CHUNK)], idx_v)
      pltpu.sync_copy(lbl_hbm.at[idx_v], tgt_v)
      pltpu.sync_copy(tgt_v, tgt_out.at[pl.ds(boff, _CHUNK)])

  return k(features, labels, lbl2d, indexes, zeros_d, zeros_c, ones_c)


_CBLK = 512


def _tc_body(x_ref, s_ref, c_ref, t_ref, o_ref, rs_acc, tv_acc):
  i = pl.program_id(0)
  x = x_ref[...]
  nrm = jnp.sqrt(jnp.sum(x * x, axis=1, keepdims=True))
  xn = x / jnp.maximum(nrm, 1e-12)
  s = s_ref[...]
  f = s[0] + s[1]
  c = c_ref[...]
  cnt = c[0, :, 0] + c[1, :, 0]
  inv = 1.0 / (_TEMP * jnp.where(cnt > 0, cnt, 1.0))
  vec = lax.dot_general(xn, f * inv[:, None], (((1,), (1,)), ((), ())),
                        preferred_element_type=jnp.float32)
  e = jnp.exp(vec) * (cnt > 0).astype(jnp.float32)[None, :]
  colid = i * _CBLK + lax.broadcasted_iota(jnp.int32, (_B, _CBLK), 1)
  tmask = (colid == t_ref[...]).astype(jnp.float32)
  ps = jnp.sum(e, axis=1, keepdims=True)
  pt = jnp.sum(e * tmask, axis=1, keepdims=True)

  @pl.when(i == 0)
  def _():
    rs_acc[...] = ps
    tv_acc[...] = pt

  @pl.when(i > 0)
  def _():
    rs_acc[...] += ps
    tv_acc[...] += pt

  @pl.when(i == pl.num_programs(0) - 1)
  def _():
    tot = rs_acc[...] + 1e-6
    logp = jnp.log(tv_acc[...] / tot + 1e-6)
    o_ref[...] = jnp.mean(-logp).reshape(1, 1)


def _tc_loss(results, sums, cnts, targets):
  return pl.pallas_call(
      _tc_body,
      grid=(_C // _CBLK,),
      in_specs=[
          pl.BlockSpec((_B, _D), lambda i: (0, 0)),
          pl.BlockSpec((2, _CBLK, _D), lambda i: (0, i, 0)),
          pl.BlockSpec((2, _CBLK, _CW), lambda i: (0, i, 0)),
          pl.BlockSpec((_B, 1), lambda i: (0, 0)),
      ],
      out_specs=pl.BlockSpec((1, 1), lambda i: (0, 0)),
      out_shape=jax.ShapeDtypeStruct((1, 1), jnp.float32),
      scratch_shapes=[
          pltpu.VMEM((_B, 1), jnp.float32),
          pltpu.VMEM((_B, 1), jnp.float32),
      ],
  )(results, sums, cnts, targets.reshape(_B, 1))


def kernel(results, indexes, features, labels):
  zeros_d = jnp.zeros((_ZROWS, _D), jnp.float32)
  zeros_c = jnp.zeros((_ZROWS, _CW), jnp.float32)
  ones_c = jnp.ones((_CHUNK, _CW), jnp.float32)
  labels = labels.astype(jnp.int32)
  lbl2d = lax.slice(labels, (0,), (_NFULL * _CHUNK,)).reshape(_NFULL, _CHUNK)
  sums, cnts, targets = _sc_segment_sum(
      features, labels, lbl2d, indexes.astype(jnp.int32),
      zeros_d, zeros_c, ones_c)
  out = _tc_loss(results, sums, cnts, targets)
  return out[0, 0]
